# baseline (speedup 1.0000x reference)
import jax
import jax.numpy as jnp
from jax.experimental import pallas as pl
from jax.experimental.pallas import tpu as pltpu

K = 8192
D = 256
N = 8192
T = 256
GRID = N // T
TPB = 1024 // T
COMMITMENT_COST = 0.25


def _vq_body(zc_ref, emb_ref, st_ref, enc_ref, loss_ref, perp_ref,
             loss_acc, hist_acc):
    i = pl.program_id(0)
    xc = zc_ref[0]
    x = jnp.transpose(xc, (1, 0))
    e = emb_ref[...]

    @pl.when(i == 0)
    def _init():
        loss_acc[0, 0] = jnp.float32(0.0)
        hist_acc[...] = jnp.zeros_like(hist_acc)

    x_norm = jnp.sum(x * x, axis=1, keepdims=True)
    e_norm = jnp.sum(e * e, axis=1)
    prod = jax.lax.dot_general(x.astype(jnp.bfloat16), e.astype(jnp.bfloat16),
                               (((1,), (1,)), ((), ())),
                               preferred_element_type=jnp.float32)
    dist = (x_norm + e_norm[None, :]) - 2.0 * prod

    mn = jnp.min(dist, axis=1, keepdims=True)
    iota = jax.lax.broadcasted_iota(jnp.int32, (T, K), 1)
    idx = jnp.min(jnp.where(dist == mn, iota, K), axis=1)
    enc = (iota == idx[:, None]).astype(jnp.float32)
    enc_ref[...] = enc

    zq = jax.lax.dot_general(enc, e, (((1,), (0,)), ((), ())))
    t = zq - x
    st_ref[0] = jnp.transpose(x + t, (1, 0))
    loss_acc[0, 0] += jnp.sum(t * t)
    hist_acc[...] += jnp.sum(enc, axis=0)[None, :]

    @pl.when(i == GRID - 1)
    def _fini():
        m = loss_acc[0, 0] / jnp.float32(N * D)
        loss_ref[0, 0] = m + COMMITMENT_COST * m
        avg = hist_acc[...] * jnp.float32(1.0 / N)
        ent = jnp.sum(avg * jnp.log(avg + 1e-10))
        perp_ref[0, 0] = jnp.exp(-ent)


def kernel(z_e, embedding):
    B, Dm, H, W = z_e.shape
    zc = z_e.reshape(B, Dm, H * W)
    st, enc, loss, perp = pl.pallas_call(
        _vq_body,
        grid=(GRID,),
        in_specs=[
            pl.BlockSpec((1, D, T), lambda i: (i // TPB, 0, i % TPB)),
            pl.BlockSpec((K, D), lambda i: (0, 0)),
        ],
        out_specs=[
            pl.BlockSpec((1, D, T), lambda i: (i // TPB, 0, i % TPB)),
            pl.BlockSpec((T, K), lambda i: (i, 0)),
            pl.BlockSpec((1, 1), lambda i: (0, 0), memory_space=pltpu.SMEM),
            pl.BlockSpec((1, 1), lambda i: (0, 0), memory_space=pltpu.SMEM),
        ],
        out_shape=[
            jax.ShapeDtypeStruct((B, Dm, H * W), jnp.float32),
            jax.ShapeDtypeStruct((N, K), jnp.float32),
            jax.ShapeDtypeStruct((1, 1), jnp.float32),
            jax.ShapeDtypeStruct((1, 1), jnp.float32),
        ],
        scratch_shapes=[
            pltpu.SMEM((1, 1), jnp.float32),
            pltpu.VMEM((1, K), jnp.float32),
        ],
    )(zc, embedding)
    out = st.reshape(B, Dm, H, W)
    return out, loss[0, 0], perp[0, 0], enc


# speedup vs baseline: 1.0760x; 1.0760x over previous
import jax
import jax.numpy as jnp
from jax.experimental import pallas as pl
from jax.experimental.pallas import tpu as pltpu

K = 8192
D = 256
N = 8192
T = 256
GRID = N // T
COMMITMENT_COST = 0.25


def _vq_body(x_ref, emb_ref, st_ref, enc_ref, loss_ref, perp_ref,
             loss_acc, hist_acc):
    i = pl.program_id(0)
    x = x_ref[...]
    e = emb_ref[...]
    @pl.when(i == 0)
    def _init():
        loss_acc[0, 0] = jnp.float32(0.0)
        hist_acc[...] = jnp.zeros_like(hist_acc)

    x_norm = jnp.sum(x * x, axis=1, keepdims=True)
    e_norm = jnp.sum(e * e, axis=1)
    x2b = (x + x).astype(jnp.bfloat16)
    eb = e.astype(jnp.bfloat16)
    prod2 = jax.lax.dot_general(x2b, eb, (((1,), (1,)), ((), ())),
                                preferred_element_type=jnp.float32)
    dist = (x_norm + e_norm[None, :]) - prod2

    mn = jnp.min(dist, axis=1, keepdims=True)
    iota = jax.lax.broadcasted_iota(jnp.int32, (T, K), 1)
    idx = jnp.min(jnp.where(dist == mn, iota, K), axis=1)
    enc = (iota == idx[:, None]).astype(jnp.float32)
    enc_ref[...] = enc

    enc_bf = enc.astype(jnp.bfloat16)

    zq = jax.lax.dot_general(enc_bf, eb, (((1,), (0,)), ((), ())),
                             preferred_element_type=jnp.float32)
    t = zq - x
    st_ref[...] = x + t
    loss_acc[0, 0] += jnp.sum(t * t)
    hist_acc[...] += jnp.sum(enc, axis=0)[None, :]

    @pl.when(i == GRID - 1)
    def _fini():
        m = loss_acc[0, 0] / jnp.float32(N * D)
        loss_ref[0, 0] = m + COMMITMENT_COST * m
        avg = hist_acc[...] * jnp.float32(1.0 / N)
        ent = jnp.sum(avg * jnp.log(avg + 1e-10))
        perp_ref[0, 0] = jnp.exp(-ent)


def kernel(z_e, embedding):
    B, Dm, H, W = z_e.shape
    z = jnp.transpose(z_e, (0, 2, 3, 1)).reshape(N, D)
    st, enc, loss, perp = pl.pallas_call(
        _vq_body,
        grid=(GRID,),
        in_specs=[
            pl.BlockSpec((T, D), lambda i: (i, 0)),
            pl.BlockSpec((K, D), lambda i: (0, 0)),
        ],
        out_specs=[
            pl.BlockSpec((T, D), lambda i: (i, 0)),
            pl.BlockSpec((T, K), lambda i: (i, 0)),
            pl.BlockSpec((1, 1), lambda i: (0, 0), memory_space=pltpu.SMEM),
            pl.BlockSpec((1, 1), lambda i: (0, 0), memory_space=pltpu.SMEM),
        ],
        out_shape=[
            jax.ShapeDtypeStruct((N, D), jnp.float32),
            jax.ShapeDtypeStruct((N, K), jnp.float32),
            jax.ShapeDtypeStruct((1, 1), jnp.float32),
            jax.ShapeDtypeStruct((1, 1), jnp.float32),
        ],
        scratch_shapes=[
            pltpu.SMEM((1, 1), jnp.float32),
            pltpu.VMEM((1, K), jnp.float32),
        ],
    )(z, embedding)
    out = jnp.transpose(st.reshape(B, H, W, Dm), (0, 3, 1, 2))
    return out, loss[0, 0], perp[0, 0], enc


# speedup vs baseline: 1.1007x; 1.0229x over previous
import functools

import jax
import jax.numpy as jnp
from jax import lax
from jax.experimental import pallas as pl
from jax.experimental.pallas import tpu as pltpu
from jax.experimental.pallas import tpu_sc as plsc

K = 8192
D = 256
N = 8192
T = 256
GRID = N // T
COMMITMENT_COST = 0.25

NC = 2
NS = 16
NW = NC * NS
BPW = N // NW
CHUNK = 64
LANES = 16


def _tc_body(x_ref, emb_ref, enc_ref, idx_ref, perp_ref, hist_acc):
    i = pl.program_id(0)
    x = x_ref[...]
    e = emb_ref[...]

    @pl.when(i == 0)
    def _init():
        hist_acc[...] = jnp.zeros_like(hist_acc)

    x_norm = jnp.sum(x * x, axis=1, keepdims=True)
    e_norm = jnp.sum(e * e, axis=1)
    x2b = (x + x).astype(jnp.bfloat16)
    eb = e.astype(jnp.bfloat16)
    prod2 = jax.lax.dot_general(x2b, eb, (((1,), (1,)), ((), ())),
                                preferred_element_type=jnp.float32)
    dist = (x_norm + e_norm[None, :]) - prod2

    mn = jnp.min(dist, axis=1, keepdims=True)
    iota = jax.lax.broadcasted_iota(jnp.int32, (T, K), 1)
    idx = jnp.min(jnp.where(dist == mn, iota, K), axis=1)
    idx_ref[...] = idx[:, None]
    enc = (iota == idx[:, None]).astype(jnp.float32)
    enc_ref[...] = enc
    hist_acc[...] += jnp.sum(enc, axis=0)[None, :]

    @pl.when(i == GRID - 1)
    def _fini():
        avg = hist_acc[...] * jnp.float32(1.0 / N)
        ent = jnp.sum(avg * jnp.log(avg + 1e-10))
        perp_ref[0, 0] = jnp.exp(-ent)


def _sc_kernel_fn(ebf_hbm, idx_hbm, x_hbm, st_hbm, part_hbm,
                  idx_v, zq_v, x_v, vec_v, sem):
    wid = lax.axis_index("s") * NC + lax.axis_index("c")
    base = wid * BPW
    pltpu.sync_copy(idx_hbm.at[pl.ds(base, BPW)], idx_v)

    acc = jnp.zeros((LANES,), jnp.float32)
    for c in range(BPW // CHUNK):
        pltpu.async_copy(ebf_hbm.at[idx_v.at[pl.ds(c * CHUNK, CHUNK)]],
                         zq_v, sem).wait()
        pltpu.sync_copy(x_hbm.at[pl.ds(base + c * CHUNK, CHUNK)], x_v)

        def row_body(r, a):
            for j in range(D // LANES):
                sl = pl.ds(j * LANES, LANES)
                zq = zq_v[r, sl]
                xv = x_v[r, sl]
                t = zq - xv
                zq_v[r, sl] = xv + t
                a = a + t * t
            return a

        acc = lax.fori_loop(0, CHUNK, row_body, acc)
        pltpu.sync_copy(zq_v, st_hbm.at[pl.ds(base + c * CHUNK, CHUNK)])

    vec_v[...] = acc
    pltpu.sync_copy(vec_v, part_hbm.at[wid])


_sc_quantize = functools.partial(
    pl.kernel,
    mesh=plsc.VectorSubcoreMesh(core_axis_name="c", subcore_axis_name="s"),
    out_type=[
        jax.ShapeDtypeStruct((N, D), jnp.float32),
        jax.ShapeDtypeStruct((NW, LANES), jnp.float32),
    ],
    scratch_types=[
        pltpu.VMEM((BPW,), jnp.int32),
        pltpu.VMEM((CHUNK, D), jnp.float32),
        pltpu.VMEM((CHUNK, D), jnp.float32),
        pltpu.VMEM((LANES,), jnp.float32),
        pltpu.SemaphoreType.DMA,
    ],
)(_sc_kernel_fn)


def kernel(z_e, embedding):
    B, Dm, H, W = z_e.shape
    z = jnp.transpose(z_e, (0, 2, 3, 1)).reshape(N, D)
    enc, idxo, perp = pl.pallas_call(
        _tc_body,
        grid=(GRID,),
        in_specs=[
            pl.BlockSpec((T, D), lambda i: (i, 0)),
            pl.BlockSpec((K, D), lambda i: (0, 0)),
        ],
        out_specs=[
            pl.BlockSpec((T, K), lambda i: (i, 0)),
            pl.BlockSpec((T, 1), lambda i: (i, 0)),
            pl.BlockSpec((1, 1), lambda i: (0, 0), memory_space=pltpu.SMEM),
        ],
        out_shape=[
            jax.ShapeDtypeStruct((N, K), jnp.float32),
            jax.ShapeDtypeStruct((N, 1), jnp.int32),
            jax.ShapeDtypeStruct((1, 1), jnp.float32),
        ],
        scratch_shapes=[
            pltpu.VMEM((1, K), jnp.float32),
        ],
    )(z, embedding)
    ebf = embedding.astype(jnp.bfloat16).astype(jnp.float32)
    st, loss_parts = _sc_quantize(ebf, idxo.reshape(N), z)
    m = jnp.sum(loss_parts) / jnp.float32(N * D)
    loss = m + COMMITMENT_COST * m
    out = jnp.transpose(st.reshape(B, H, W, Dm), (0, 3, 1, 2))
    return out, loss, perp[0, 0], enc
